# baseline (speedup 1.0000x reference)
import functools

import jax
import jax.numpy as jnp
from jax.experimental import pallas as pl
from jax.experimental.pallas import tpu as pltpu

BL = 512
NBUF = 4


def _body(n_ref, bagf_ref, blkf_ref, bagc_ref, blkc_ref, tot_ref,
          x_ref, w1_ref, b1_ref, w2_ref, b2_ref, out_ref,
          xbuf, xb16a, xb16b, acc_ref, sems, *, bl: int):
    pairs = tot_ref[0]

    def issue(t):
        slot = jax.lax.rem(t, NBUF)
        b = bagf_ref[t]
        jj = blkf_ref[t]
        pltpu.make_async_copy(
            x_ref.at[b, pl.ds(jj * bl, bl), :], xbuf.at[slot],
            sems.at[slot]).start()

    def wait(t):
        slot = jax.lax.rem(t, NBUF)
        b = bagf_ref[t]
        jj = blkf_ref[t]
        pltpu.make_async_copy(
            x_ref.at[b, pl.ds(jj * bl, bl), :], xbuf.at[slot],
            sems.at[slot]).wait()

    for t0 in range(NBUF):
        issue(t0)

    wait(0)
    xb16a[...] = xbuf[0].astype(jnp.bfloat16)

    def half_step(t, cur_ref, nxt_ref):
        wait(t + 1)
        issue(t + NBUF)
        nxt_ref[...] = xbuf[jax.lax.rem(t + 1, NBUF)].astype(jnp.bfloat16)

        b = bagc_ref[t]
        jj = blkc_ref[t]
        nb = n_ref[b]

        h = jnp.dot(cur_ref[...], w1_ref[...],
                    preferred_element_type=jnp.float32)
        h = jnp.maximum(h + b1_ref[...], 0.0)
        rows = jax.lax.broadcasted_iota(jnp.int32, (bl, 1), 0) + jj * bl
        h = jnp.where(rows < nb, h, 0.0)
        red = jnp.sum(h.reshape(bl // 8, 8, -1), axis=0)
        acc_ref[...] = jnp.where(jj == 0, red, acc_ref[...] + red)

        pooled = jnp.sum(acc_ref[...], axis=0, keepdims=True)
        pooled = pooled / nb.astype(jnp.float32)
        res = jnp.dot(pooled, w2_ref[...],
                      preferred_element_type=jnp.float32) + b2_ref[...]
        out_ref[b] = res

    def step(p, carry):
        t = 2 * p
        half_step(t, xb16a, xb16b)
        half_step(t + 1, xb16b, xb16a)
        return carry

    jax.lax.fori_loop(0, pairs, step, 0)

    for k in range(1, NBUF):
        wait(2 * pairs + k)


def kernel(x, n_instances, W1, b1, W2, b2):
    B, L, D = x.shape
    DO = W2.shape[1]
    nj = L // BL
    n32 = n_instances.astype(jnp.int32)

    nblk = (n32 + BL - 1) // BL
    ends = jnp.cumsum(nblk)
    starts = ends - nblk
    total = ends[-1]
    pairs = (total + 1) // 2
    tpad = B * nj + 2 * NBUF + 4
    t_idx = jnp.arange(tpad, dtype=jnp.int32)
    bag_raw = jnp.searchsorted(ends, t_idx, side="right").astype(jnp.int32)
    bagf = jnp.where(t_idx < total, jnp.minimum(bag_raw, B - 1), 0)
    blkf = jnp.where(t_idx < total, t_idx - starts[bagf], 0)
    bagc = jnp.where(t_idx < total, bagf, B)
    blkc = blkf
    n_pad = jnp.concatenate([n32, jnp.ones((1,), jnp.int32)])

    grid_spec = pltpu.PrefetchScalarGridSpec(
        num_scalar_prefetch=6,
        grid=(1,),
        in_specs=[
            pl.BlockSpec(memory_space=pl.ANY),
            pl.BlockSpec((D, D), lambda i, *_: (0, 0)),
            pl.BlockSpec((1, D), lambda i, *_: (0, 0)),
            pl.BlockSpec((D, DO), lambda i, *_: (0, 0)),
            pl.BlockSpec((1, DO), lambda i, *_: (0, 0)),
        ],
        out_specs=pl.BlockSpec((B + 1, 1, DO), lambda i, *_: (0, 0, 0)),
        scratch_shapes=[
            pltpu.VMEM((NBUF, BL, D), jnp.float32),
            pltpu.VMEM((BL, D), jnp.bfloat16),
            pltpu.VMEM((BL, D), jnp.bfloat16),
            pltpu.VMEM((8, D), jnp.float32),
            pltpu.SemaphoreType.DMA((NBUF,)),
        ],
    )

    out = pl.pallas_call(
        functools.partial(_body, bl=BL),
        grid_spec=grid_spec,
        out_shape=jax.ShapeDtypeStruct((B + 1, 1, DO), jnp.float32),
    )(n_pad, bagf, blkf, bagc, blkc, pairs.reshape(1), x,
      W1.astype(jnp.bfloat16), b1.reshape(1, D), W2, b2.reshape(1, DO))
    return out[:B].reshape(B, DO)
